# Initial kernel scaffold; baseline (speedup 1.0000x reference)
import jax
import jax.numpy as jnp
from jax.experimental import pallas as pl


def kernel(input_atom, input_bond, atom_graph, bond_graph, num_nbs, node_mask, W_atom, W_nei_atom, b_nei_atom, W_nei_bond, b_nei_bond, W_self, b_self, W_U2, b_U2, W_U1, b_U1, W_out, b_out):
    raise NotImplementedError("write your pallas kernel here")


# speedup vs baseline: 26.0285x; 26.0285x over previous
import jax
import jax.numpy as jnp
from jax import lax
from jax.experimental import pallas as pl
from jax.experimental.pallas import tpu as pltpu

B, A, MAXNB, BONDS = 512, 120, 10, 512
AFD, BFD, H = 82, 6, 128
DEPTH = 3
S = A * MAXNB


def _wln_body(ia_ref, ib_ref, an_ref, bn_ref, nn_ref, nm_ref,
              W_atom_ref, W_nei_atom_ref, b_nei_atom_ref,
              W_nei_bond_ref, b_nei_bond_ref,
              W_self_ref, b_self_ref,
              W_U2a_ref, W_U2b_ref, b_U2_ref,
              W_U1a_ref, W_U1b_ref, b_U1_ref,
              W_out_ref, b_out_ref, out_ref):
    f32 = jnp.float32
    ia = ia_ref[0]
    ib = ib_ref[0]
    an = an_ref[0]
    bn = bn_ref[0]
    nn = nn_ref[0]
    nm = nm_ref[0]

    j_of_slot = lax.broadcasted_iota(jnp.int32, (S, 1), 0) // A
    maskf = (j_of_slot < nn).astype(f32)

    acols = lax.broadcasted_iota(jnp.int32, (S, A), 1)
    bcols = lax.broadcasted_iota(jnp.int32, (S, BONDS), 1)
    Pa = (acols == an).astype(f32)
    Pb = (bcols == bn).astype(f32)

    def mm(x, w):
        return jnp.dot(x, w, preferred_element_type=f32)

    fbond = mm(Pb, ib)
    bu2 = mm(fbond, W_U2b_ref[...]) + b_U2_ref[...]

    def seg_sum(x):
        acc = x[0:A, :]
        for j in range(1, MAXNB):
            acc = acc + x[j * A:(j + 1) * A, :]
        return acc

    af = jnp.maximum(mm(ia, W_atom_ref[...]), 0.0)

    for _ in range(DEPTH - 1):
        au2 = mm(af, W_U2a_ref[...])
        pre = jnp.maximum(mm(Pa, au2) + bu2, 0.0) * maskf
        nei_label = seg_sum(pre)
        af = mm(af, W_U1a_ref[...]) + mm(nei_label, W_U1b_ref[...]) + b_U1_ref[...]

    ha = mm(af, W_nei_atom_ref[...]) + b_nei_atom_ref[...]
    hb_nei = mm(fbond, W_nei_bond_ref[...]) + b_nei_bond_ref[...]
    prod = mm(Pa, ha) * hb_nei * maskf
    f_nei = seg_sum(prod)
    f_self = mm(af, W_self_ref[...]) + b_self_ref[...]
    kern = f_nei * f_self * nm
    x = jnp.sum(kern, axis=0, keepdims=True)
    res = mm(x, W_out_ref[...]) + b_out_ref[...]
    out_ref[...] = res.reshape(1, 1, 1)


def kernel(input_atom, input_bond, atom_graph, bond_graph, num_nbs, node_mask,
           W_atom, W_nei_atom, b_nei_atom, W_nei_bond, b_nei_bond,
           W_self, b_self, W_U2, b_U2, W_U1, b_U1, W_out, b_out):
    f32 = jnp.float32
    a_nei = jnp.transpose(atom_graph[..., 1], (0, 2, 1)).reshape(B, S, 1)
    b_nei = jnp.transpose(bond_graph[..., 1], (0, 2, 1)).reshape(B, S, 1)
    nn_rep = jnp.broadcast_to(num_nbs[:, None, :], (B, MAXNB, A)).reshape(B, S, 1)

    W_U2a, W_U2b = W_U2[:H], W_U2[H:]
    W_U1a, W_U1b = W_U1[:H], W_U1[H:]

    def vec(b):
        return b.reshape(1, -1).astype(f32)

    grid = (B,)
    full = lambda shape: pl.BlockSpec(shape, lambda i: (0,) * len(shape))
    perm = lambda shape: pl.BlockSpec((1,) + shape, lambda i: (i,) + (0,) * len(shape))

    out = pl.pallas_call(
        _wln_body,
        grid=grid,
        in_specs=[
            perm((A, AFD)),
            perm((BONDS, BFD)),
            perm((S, 1)),
            perm((S, 1)),
            perm((S, 1)),
            perm((A, 1)),
            full((AFD, H)),
            full((H, H)),
            full((1, H)),
            full((BFD, H)),
            full((1, H)),
            full((H, H)),
            full((1, H)),
            full((H, H)),
            full((BFD, H)),
            full((1, H)),
            full((H, H)),
            full((H, H)),
            full((1, H)),
            full((H, 1)),
            full((1, 1)),
        ],
        out_specs=pl.BlockSpec((1, 1, 1), lambda i: (i, 0, 0)),
        out_shape=jax.ShapeDtypeStruct((B, 1, 1), f32),
    )(input_atom, input_bond, a_nei, b_nei, nn_rep, node_mask,
      W_atom, W_nei_atom, vec(b_nei_atom), W_nei_bond, vec(b_nei_bond),
      W_self, vec(b_self), W_U2a, W_U2b, vec(b_U2),
      W_U1a, W_U1b, vec(b_U1), W_out, vec(b_out))
    return out.reshape(B, 1)
